# baseline (speedup 1.0000x reference)
import functools

import jax
import jax.numpy as jnp
from jax import lax
from jax.experimental import pallas as pl
from jax.experimental.pallas import tpu as pltpu
from jax.experimental.pallas import tpu_sc as plsc

D = 128
H = 4
DH = 32
KNN = 16
TOPK = 8
NC, NS, LANES = 2, 16, 16
NW = NC * NS
BIG = 3.0e38


def _shuf(x, s):
    perm = jnp.arange(LANES, dtype=jnp.int32) ^ s
    return x.at[perm].get(mode="promise_in_bounds")


def _bmin(x):
    for s in (8, 4, 2, 1):
        x = jnp.minimum(x, _shuf(x, s))
    return x


def _bmax(x):
    for s in (8, 4, 2, 1):
        x = jnp.maximum(x, _shuf(x, s))
    return x


def _bsum(x):
    for s in (8, 4, 2, 1):
        x = x + _shuf(x, s)
    return x


def _bf16r(x):
    u = plsc.bitcast(x, jnp.int32)
    r = (u + 0x7FFF + ((u >> 16) & 1)) & jnp.int32(-65536)
    return plsc.bitcast(r, jnp.float32)


def _d2_body(cfull_ref, crow_ref, out_ref):
    c = cfull_ref[0]
    cr = crow_ref[0]
    sq = jnp.sum(c * c, axis=0, keepdims=True)
    sqr = jnp.sum(cr * cr, axis=0, keepdims=True)
    dot = lax.dot_general(cr.astype(jnp.bfloat16), c.astype(jnp.bfloat16),
                          (((0,), (0,)), ((), ())),
                          preferred_element_type=jnp.float32)
    d2 = sqr.reshape(-1, 1) + sq - 2.0 * dot
    out_ref[0] = jnp.sqrt(jnp.maximum(d2, 1e-12))


def _d2_call(coords_t, BR=256):
    B, _, V = coords_t.shape
    return pl.pallas_call(
        _d2_body,
        grid=(B, V // BR),
        in_specs=[
            pl.BlockSpec((1, 3, V), lambda b, t: (b, 0, 0)),
            pl.BlockSpec((1, 3, BR), lambda b, t: (b, 0, t)),
        ],
        out_specs=pl.BlockSpec((1, BR, V), lambda b, t: (b, t, 0)),
        out_shape=jax.ShapeDtypeStruct((B, V, V), jnp.float32),
    )(coords_t, coords_t)


def _proj_body(x_ref, w_ref, b_ref, a_ref, n_ref, q_ref, k_ref, v_ref):
    y = lax.dot_general(x_ref[...].astype(jnp.bfloat16),
                        w_ref[...].astype(jnp.bfloat16),
                        (((1,), (0,)), ((), ())),
                        preferred_element_type=jnp.float32)
    y = y + b_ref[...]
    a_ref[...] = y[:, 0:64]
    n_ref[...] = y[:, 64:128]
    q_ref[...] = y[:, 128:256]
    k_ref[...] = y[:, 256:384]
    v_ref[...] = y[:, 384:512]


def _proj_call(x2, wcat, bcat, BR=512):
    R = x2.shape[0]
    outs = [
        jax.ShapeDtypeStruct((R, 64), jnp.float32),
        jax.ShapeDtypeStruct((R, 64), jnp.float32),
        jax.ShapeDtypeStruct((R, D), jnp.float32),
        jax.ShapeDtypeStruct((R, D), jnp.float32),
        jax.ShapeDtypeStruct((R, D), jnp.float32),
    ]
    return pl.pallas_call(
        _proj_body,
        grid=(R // BR,),
        in_specs=[
            pl.BlockSpec((BR, D), lambda i: (i, 0)),
            pl.BlockSpec((D, 512), lambda i: (0, 0)),
            pl.BlockSpec((1, 512), lambda i: (0, 0)),
        ],
        out_specs=[
            pl.BlockSpec((BR, 64), lambda i: (i, 0)),
            pl.BlockSpec((BR, 64), lambda i: (i, 0)),
            pl.BlockSpec((BR, D), lambda i: (i, 0)),
            pl.BlockSpec((BR, D), lambda i: (i, 0)),
            pl.BlockSpec((BR, D), lambda i: (i, 0)),
        ],
        out_shape=outs,
    )(x2, wcat, bcat)


def _outp_body(x_ref, w_ref, b_ref, m_ref, o_ref):
    y = lax.dot_general(x_ref[...], w_ref[...], (((1,), (0,)), ((), ())))
    o_ref[...] = (y + b_ref[...]) * m_ref[...]


def _outp_call(attn, wo, bo2, maskf, BR=512):
    R = attn.shape[0]
    return pl.pallas_call(
        _outp_body,
        grid=(R // BR,),
        in_specs=[
            pl.BlockSpec((BR, D), lambda i: (i, 0)),
            pl.BlockSpec((D, D), lambda i: (0, 0)),
            pl.BlockSpec((1, D), lambda i: (0, 0)),
            pl.BlockSpec((BR, 1), lambda i: (i, 0)),
        ],
        out_specs=pl.BlockSpec((BR, D), lambda i: (i, 0)),
        out_shape=jax.ShapeDtypeStruct((R, D), jnp.float32),
    )(attn, wo, bo2, maskf)


def _sc_sparse(d2, aproj, nproj, qarr, karr, varr, coords, w1p, w2):
    R, V = d2.shape
    B = coords.shape[0]
    rows_per_w = R // NW
    rows_per_b = R // B
    nblk = V // LANES
    mesh = plsc.VectorSubcoreMesh(
        core_axis_name="c", subcore_axis_name="s",
        num_cores=NC, num_subcores=NS)

    @functools.partial(
        pl.kernel,
        out_type=jax.ShapeDtypeStruct((R, D), jnp.float32),
        mesh=mesh,
        compiler_params=pltpu.CompilerParams(
            needs_layout_passes=False, use_tc_tiling_on_sc=False),
        scratch_types=[
            pltpu.VMEM((V,), jnp.float32),
            pltpu.VMEM((V, 3), jnp.float32),
            pltpu.VMEM((64,), jnp.float32),
            pltpu.VMEM((D,), jnp.float32),
            pltpu.VMEM((KNN, 64), jnp.float32),
            pltpu.VMEM((TOPK, D), jnp.float32),
            pltpu.VMEM((TOPK, D), jnp.float32),
            pltpu.VMEM((32,), jnp.int32),
            pltpu.VMEM((KNN,), jnp.int32),
            pltpu.VMEM((KNN,), jnp.int32),
            pltpu.VMEM((D,), jnp.float32),
            pltpu.VMEM((3, 64), jnp.float32),
            pltpu.VMEM((64,), jnp.float32),
            pltpu.SemaphoreType.DMA,
            pltpu.SemaphoreType.DMA,
        ],
    )
    def run(d2_hbm, a_hbm, n_hbm, q_hbm, k_hbm, v_hbm, c_hbm, w1p_hbm,
            w2_hbm, out_hbm, d2row, coords_vm, a_vm, q_vm, nbuf, kbuf,
            vbuf, knnbuf, gix_vm, top8_vm, outbuf, w1p_vm, w2_vm,
            sem1, sem2):
        wid = lax.axis_index("s") * NC + lax.axis_index("c")
        base = wid * rows_per_w
        b = base // rows_per_b
        lane = lax.iota(jnp.int32, LANES)
        lane0 = lane == 0
        lane8 = lane < TOPK
        lanemod8 = lane & 7
        inv_sqrt_dh = jnp.float32(1.0 / (DH ** 0.5))

        pltpu.sync_copy(c_hbm.at[b], coords_vm)
        pltpu.sync_copy(w1p_hbm, w1p_vm)
        pltpu.sync_copy(w2_hbm, w2_vm)

        def anchor_body(ii, _carry):
            r = base + ii
            i = r - b * rows_per_b
            pltpu.sync_copy(d2_hbm.at[r], d2row)
            pltpu.sync_copy(a_hbm.at[r], a_vm)
            pltpu.sync_copy(q_hbm.at[r], q_vm)

            col = d2row[pl.ds(0, LANES)]
            colv = jnp.zeros((LANES,), jnp.int32)
            for v in range(1, nblk):
                blk = d2row[pl.ds(v * LANES, LANES)]
                lt = blk < col
                col = jnp.where(lt, blk, col)
                colv = jnp.where(lt, v, colv)

            for rd in range(KNN + 1):
                m = _bmin(col)
                jcand = jnp.where(col == m, colv * LANES + lane,
                                  jnp.int32(1 << 30))
                j = _bmin(jcand)
                slot = 16 if rd == 0 else rd - 1
                plsc.store_scatter(knnbuf, [jnp.full((LANES,), slot, jnp.int32)],
                                   j, mask=lane0)
                plsc.store_scatter(d2row, [j], jnp.full((LANES,), BIG),
                                   mask=lane0)
                if rd == KNN:
                    break
                lstar = j & (LANES - 1)
                base_jj = lane * LANES + lstar
                gs = [plsc.load_gather(d2row, [t * 256 + base_jj])
                      for t in range(nblk // LANES)]
                cmv = gs[0]
                for t in range(1, nblk // LANES):
                    cmv = jnp.minimum(cmv, gs[t])
                colmin = _bmin(cmv)
                bestv = jnp.full((LANES,), 4096, jnp.int32)
                for t in range(nblk // LANES):
                    first = plsc.all_reduce_ffs(gs[t] == colmin)
                    bestv = jnp.where((bestv >= 4096) & (first < LANES),
                                      t * LANES + first, bestv)
                sel = lane == lstar
                col = jnp.where(sel, colmin, col)
                colv = jnp.where(sel, bestv, colv)
            knn = knnbuf[pl.ds(0, KNN)]
            gix_vm[...] = knn + b * rows_per_b

            pltpu.async_copy(n_hbm.at[gix_vm], nbuf, sem1).wait()

            ivec = jnp.broadcast_to(i, (LANES,))
            nx = plsc.load_gather(coords_vm, [knn, jnp.zeros((LANES,), jnp.int32)])
            ny = plsc.load_gather(coords_vm, [knn, jnp.full((LANES,), 1, jnp.int32)])
            nz = plsc.load_gather(coords_vm, [knn, jnp.full((LANES,), 2, jnp.int32)])
            dx = _bf16r(nx - plsc.load_gather(coords_vm, [ivec, jnp.zeros((LANES,), jnp.int32)]))
            dy = _bf16r(ny - plsc.load_gather(coords_vm, [ivec, jnp.full((LANES,), 1, jnp.int32)]))
            dz = _bf16r(nz - plsc.load_gather(coords_vm, [ivec, jnp.full((LANES,), 2, jnp.int32)]))

            a_vecs = [a_vm[pl.ds(g * LANES, LANES)] for g in range(4)]
            w2_vecs = [w2_vm[pl.ds(g * LANES, LANES)] for g in range(4)]
            wpx = [w1p_vm[0, pl.ds(g * LANES, LANES)] for g in range(4)]
            wpy = [w1p_vm[1, pl.ds(g * LANES, LANES)] for g in range(4)]
            wpz = [w1p_vm[2, pl.ds(g * LANES, LANES)] for g in range(4)]
            scores = jnp.zeros((LANES,))
            for c in range(64):
                g, e = c // LANES, c % LANES
                ncol = plsc.load_gather(nbuf, [lane, jnp.full((LANES,), c, jnp.int32)])
                rp = dx * wpx[g][e] + dy * wpy[g][e] + dz * wpz[g][e]
                hh = jnp.maximum(ncol + rp + a_vecs[g][e], 0.0)
                scores = scores + _bf16r(hh) * w2_vecs[g][e]

            _, srt = plsc.sort_key_val(scores, gix_vm[...], descending=True)
            top8_vm[...] = srt
            idx8 = top8_vm.at[pl.ds(0, TOPK)]
            ck = pltpu.async_copy(k_hbm.at[idx8], kbuf, sem1)
            cv = pltpu.async_copy(v_hbm.at[idx8], vbuf, sem2)
            ck.wait()
            cv.wait()

            q_vecs = [q_vm[pl.ds(g * LANES, LANES)] for g in range(D // LANES)]
            ws = []
            for h in range(H):
                s8 = jnp.zeros((LANES,))
                for c in range(DH):
                    cc = h * DH + c
                    kcol = plsc.load_gather(
                        kbuf, [lanemod8, jnp.full((LANES,), cc, jnp.int32)])
                    s8 = s8 + kcol * q_vecs[cc // LANES][cc % LANES]
                s8 = s8 * inv_sqrt_dh
                mx = _bmax(jnp.where(lane8, s8, -BIG))
                e = jnp.where(lane8, jnp.exp(s8 - mx), 0.0)
                ws.append(e / _bsum(e))

            for h in range(H):
                for blk in range(DH // LANES):
                    off = h * DH + blk * LANES
                    acc = jnp.zeros((LANES,))
                    for kk in range(TOPK):
                        acc = acc + ws[h][kk] * vbuf[kk, pl.ds(off, LANES)]
                    outbuf[pl.ds(off, LANES)] = acc
            pltpu.sync_copy(outbuf, out_hbm.at[r])
            return _carry

        lax.fori_loop(0, rows_per_w, anchor_body, 0)

    return run(d2, aproj, nproj, qarr, karr, varr, coords, w1p, w2)


@jax.jit
def kernel(voxel_tokens, voxel_coords, mask, es_w1, es_b1, es_w2, es_b2,
           wq, bq, wk, bk, wv, bv, wo, bo):
    B, V, _ = voxel_tokens.shape
    R = B * V
    coords_t = jnp.swapaxes(voxel_coords, 1, 2)
    d2 = _d2_call(coords_t).reshape(R, V)

    x2 = voxel_tokens.reshape(R, D)
    wcat = jnp.concatenate(
        [es_w1[:D], es_w1[D:2 * D], wq, wk, wv], axis=1)
    bcat = jnp.concatenate(
        [es_b1, jnp.zeros((64,), jnp.float32), bq, bk, bv]).reshape(1, 512)
    aproj, nproj, qarr, karr, varr = _proj_call(x2, wcat, bcat)

    def _rnd_bf16(x):
        u = lax.bitcast_convert_type(x, jnp.int32)
        r = (u + 0x7FFF + ((u >> 16) & 1)) & jnp.int32(-65536)
        return lax.bitcast_convert_type(r, jnp.float32)
    w1p = _rnd_bf16(es_w1[2 * D:])
    w2 = _rnd_bf16(es_w2[:, 0])
    attn = _sc_sparse(d2, aproj, nproj, qarr, karr, varr,
                      voxel_coords, w1p, w2)

    maskf = mask.reshape(R, 1).astype(jnp.float32)
    out = _outp_call(attn, wo, bo.reshape(1, D), maskf)
    return out.reshape(B, V, D)


# speedup vs baseline: 1.4281x; 1.4281x over previous
import functools

import jax
import jax.numpy as jnp
from jax import lax
from jax.experimental import pallas as pl
from jax.experimental.pallas import tpu as pltpu
from jax.experimental.pallas import tpu_sc as plsc

D = 128
H = 4
DH = 32
KNN = 16
TOPK = 8
NC, NS, LANES = 2, 16, 16
NW = NC * NS
BIG = 3.0e38


def _shuf(x, s):
    perm = jnp.arange(LANES, dtype=jnp.int32) ^ s
    return x.at[perm].get(mode="promise_in_bounds")


def _bmin(x):
    for s in (8, 4, 2, 1):
        x = jnp.minimum(x, _shuf(x, s))
    return x


def _bmax(x):
    for s in (8, 4, 2, 1):
        x = jnp.maximum(x, _shuf(x, s))
    return x


def _bsum(x):
    for s in (8, 4, 2, 1):
        x = x + _shuf(x, s)
    return x


def _bf16r(x):
    u = plsc.bitcast(x, jnp.int32)
    r = (u + 0x7FFF + ((u >> 16) & 1)) & jnp.int32(-65536)
    return plsc.bitcast(r, jnp.float32)


def _d2_body(cfull_ref, crow_ref, out_ref):
    c = cfull_ref[0]
    cr = crow_ref[0]
    sq = jnp.sum(c * c, axis=0, keepdims=True)
    sqr = jnp.sum(cr * cr, axis=0, keepdims=True)
    dot = lax.dot_general(cr.astype(jnp.bfloat16), c.astype(jnp.bfloat16),
                          (((0,), (0,)), ((), ())),
                          preferred_element_type=jnp.float32)
    d2 = sqr.reshape(-1, 1) + sq - 2.0 * dot
    out_ref[0] = jnp.sqrt(jnp.maximum(d2, 1e-12))


def _d2_call(coords_t, BR=256):
    B, _, V = coords_t.shape
    return pl.pallas_call(
        _d2_body,
        grid=(B, V // BR),
        in_specs=[
            pl.BlockSpec((1, 3, V), lambda b, t: (b, 0, 0)),
            pl.BlockSpec((1, 3, BR), lambda b, t: (b, 0, t)),
        ],
        out_specs=pl.BlockSpec((1, BR, V), lambda b, t: (b, t, 0)),
        out_shape=jax.ShapeDtypeStruct((B, V, V), jnp.float32),
    )(coords_t, coords_t)


def _proj_body(x_ref, w_ref, b_ref, a_ref, n_ref, q_ref, k_ref, v_ref):
    y = lax.dot_general(x_ref[...].astype(jnp.bfloat16),
                        w_ref[...].astype(jnp.bfloat16),
                        (((1,), (0,)), ((), ())),
                        preferred_element_type=jnp.float32)
    y = y + b_ref[...]
    a_ref[...] = y[:, 0:64]
    n_ref[...] = y[:, 64:128]
    q_ref[...] = y[:, 128:256]
    k_ref[...] = y[:, 256:384]
    v_ref[...] = y[:, 384:512]


def _proj_call(x2, wcat, bcat, BR=512):
    R = x2.shape[0]
    outs = [
        jax.ShapeDtypeStruct((R, 64), jnp.float32),
        jax.ShapeDtypeStruct((R, 64), jnp.float32),
        jax.ShapeDtypeStruct((R, D), jnp.float32),
        jax.ShapeDtypeStruct((R, D), jnp.float32),
        jax.ShapeDtypeStruct((R, D), jnp.float32),
    ]
    return pl.pallas_call(
        _proj_body,
        grid=(R // BR,),
        in_specs=[
            pl.BlockSpec((BR, D), lambda i: (i, 0)),
            pl.BlockSpec((D, 512), lambda i: (0, 0)),
            pl.BlockSpec((1, 512), lambda i: (0, 0)),
        ],
        out_specs=[
            pl.BlockSpec((BR, 64), lambda i: (i, 0)),
            pl.BlockSpec((BR, 64), lambda i: (i, 0)),
            pl.BlockSpec((BR, D), lambda i: (i, 0)),
            pl.BlockSpec((BR, D), lambda i: (i, 0)),
            pl.BlockSpec((BR, D), lambda i: (i, 0)),
        ],
        out_shape=outs,
    )(x2, wcat, bcat)


def _outp_body(x_ref, w_ref, b_ref, m_ref, o_ref):
    y = lax.dot_general(x_ref[...], w_ref[...], (((1,), (0,)), ((), ())))
    o_ref[...] = (y + b_ref[...]) * m_ref[...]


def _outp_call(attn, wo, bo2, maskf, BR=512):
    R = attn.shape[0]
    return pl.pallas_call(
        _outp_body,
        grid=(R // BR,),
        in_specs=[
            pl.BlockSpec((BR, D), lambda i: (i, 0)),
            pl.BlockSpec((D, D), lambda i: (0, 0)),
            pl.BlockSpec((1, D), lambda i: (0, 0)),
            pl.BlockSpec((BR, 1), lambda i: (i, 0)),
        ],
        out_specs=pl.BlockSpec((BR, D), lambda i: (i, 0)),
        out_shape=jax.ShapeDtypeStruct((R, D), jnp.float32),
    )(attn, wo, bo2, maskf)


def _sc_sparse(d2, aproj, nproj, qarr, karr, varr, coords, w1p, w2):
    R, V = d2.shape
    B = coords.shape[0]
    rows_per_w = R // NW
    rows_per_b = R // B
    nblk = V // LANES
    mesh = plsc.VectorSubcoreMesh(
        core_axis_name="c", subcore_axis_name="s",
        num_cores=NC, num_subcores=NS)

    @functools.partial(
        pl.kernel,
        out_type=jax.ShapeDtypeStruct((R, D), jnp.float32),
        mesh=mesh,
        compiler_params=pltpu.CompilerParams(
            needs_layout_passes=False, use_tc_tiling_on_sc=False),
        scratch_types=[
            pltpu.VMEM((V,), jnp.float32),
            pltpu.VMEM((V,), jnp.float32),
            pltpu.VMEM((V, 3), jnp.float32),
            pltpu.VMEM((R // NW, 64), jnp.float32),
            pltpu.VMEM((R // NW, D), jnp.float32),
            pltpu.VMEM((KNN, 64), jnp.float32),
            pltpu.VMEM((TOPK, D), jnp.float32),
            pltpu.VMEM((TOPK, D), jnp.float32),
            pltpu.VMEM((32,), jnp.int32),
            pltpu.VMEM((KNN,), jnp.int32),
            pltpu.VMEM((KNN,), jnp.int32),
            pltpu.VMEM((D,), jnp.float32),
            pltpu.VMEM((3, 64), jnp.float32),
            pltpu.VMEM((64,), jnp.float32),
            pltpu.SemaphoreType.DMA,
            pltpu.SemaphoreType.DMA,
            pltpu.SemaphoreType.DMA,
            pltpu.SemaphoreType.DMA,
            pltpu.SemaphoreType.DMA,
        ],
    )
    def run(d2_hbm, a_hbm, n_hbm, q_hbm, k_hbm, v_hbm, c_hbm, w1p_hbm,
            w2_hbm, out_hbm, d2rowA, d2rowB, coords_vm, ablock, qblock,
            nbuf, kbuf, vbuf, knnbuf, gix_vm, top8_vm, outbuf, w1p_vm,
            w2_vm, semA, semB, semN, semK, semV):
        wid = lax.axis_index("s") * NC + lax.axis_index("c")
        base = wid * rows_per_w
        b = base // rows_per_b
        lane = lax.iota(jnp.int32, LANES)
        lane0 = lane == 0
        lane8 = lane < TOPK
        lanemod8 = lane & 7
        inv_sqrt_dh = jnp.float32(1.0 / (DH ** 0.5))

        pltpu.async_copy(d2_hbm.at[base], d2rowA, semA)
        pltpu.sync_copy(c_hbm.at[b], coords_vm)
        pltpu.sync_copy(w1p_hbm, w1p_vm)
        pltpu.sync_copy(w2_hbm, w2_vm)
        pltpu.sync_copy(a_hbm.at[pl.ds(base, rows_per_w)], ablock)
        pltpu.sync_copy(q_hbm.at[pl.ds(base, rows_per_w)], qblock)

        w2_vecs = [w2_vm[pl.ds(g * LANES, LANES)] for g in range(4)]
        wpx = [w1p_vm[0, pl.ds(g * LANES, LANES)] for g in range(4)]
        wpy = [w1p_vm[1, pl.ds(g * LANES, LANES)] for g in range(4)]
        wpz = [w1p_vm[2, pl.ds(g * LANES, LANES)] for g in range(4)]

        def process(r, ii, d2row):
            i = r - b * rows_per_b

            def fold(v, carry):
                col, colv = carry
                blk = d2row[pl.ds(pl.multiple_of(v * LANES, LANES), LANES)]
                lt = blk < col
                return jnp.where(lt, blk, col), jnp.where(lt, v, colv)
            col, colv = lax.fori_loop(
                0, nblk, fold,
                (jnp.full((LANES,), BIG), jnp.zeros((LANES,), jnp.int32)))

            def rnd(rd, carry):
                col, colv = carry
                m = _bmin(col)
                jcand = jnp.where(col == m, colv * LANES + lane,
                                  jnp.int32(1 << 30))
                j = _bmin(jcand)
                slot = jnp.where(rd == 0, 16, rd - 1)
                plsc.store_scatter(knnbuf, [jnp.broadcast_to(slot, (LANES,))],
                                   j, mask=lane0)
                plsc.store_scatter(d2row, [j], jnp.full((LANES,), BIG),
                                   mask=lane0)
                lstar = j & (LANES - 1)
                base_jj = lane * LANES + lstar
                gs = [plsc.load_gather(d2row, [t * 256 + base_jj])
                      for t in range(nblk // LANES)]
                cmv = gs[0]
                for t in range(1, nblk // LANES):
                    cmv = jnp.minimum(cmv, gs[t])
                colmin = _bmin(cmv)
                bestv = jnp.full((LANES,), 4096, jnp.int32)
                for t in range(nblk // LANES):
                    first = plsc.all_reduce_ffs(gs[t] == colmin)
                    bestv = jnp.where((bestv >= 4096) & (first < LANES),
                                      t * LANES + first, bestv)
                sel = lane == lstar
                return (jnp.where(sel, colmin, col),
                        jnp.where(sel, bestv, colv))
            col, colv = lax.fori_loop(0, KNN, rnd, (col, colv))
            m = _bmin(col)
            jcand = jnp.where(col == m, colv * LANES + lane, jnp.int32(1 << 30))
            j = _bmin(jcand)
            plsc.store_scatter(knnbuf, [jnp.full((LANES,), 15, jnp.int32)],
                               j, mask=lane0)

            knn = knnbuf[pl.ds(0, KNN)]
            gix_vm[...] = knn + b * rows_per_b

            cn = pltpu.async_copy(n_hbm.at[gix_vm], nbuf, semN)

            ivec = jnp.broadcast_to(i, (LANES,))
            nx = plsc.load_gather(coords_vm, [knn, jnp.zeros((LANES,), jnp.int32)])
            ny = plsc.load_gather(coords_vm, [knn, jnp.full((LANES,), 1, jnp.int32)])
            nz = plsc.load_gather(coords_vm, [knn, jnp.full((LANES,), 2, jnp.int32)])
            dx = _bf16r(nx - plsc.load_gather(coords_vm, [ivec, jnp.zeros((LANES,), jnp.int32)]))
            dy = _bf16r(ny - plsc.load_gather(coords_vm, [ivec, jnp.full((LANES,), 1, jnp.int32)]))
            dz = _bf16r(nz - plsc.load_gather(coords_vm, [ivec, jnp.full((LANES,), 2, jnp.int32)]))
            a_vecs = [ablock[ii, pl.ds(g * LANES, LANES)] for g in range(4)]
            rps = []
            for c in range(64):
                g, e = c // LANES, c % LANES
                rps.append(dx * wpx[g][e] + dy * wpy[g][e] + dz * wpz[g][e])
            cn.wait()

            scores = jnp.zeros((LANES,))
            for c in range(64):
                g, e = c // LANES, c % LANES
                ncol = plsc.load_gather(nbuf, [lane, jnp.full((LANES,), c, jnp.int32)])
                hh = jnp.maximum(ncol + rps[c] + a_vecs[g][e], 0.0)
                scores = scores + _bf16r(hh) * w2_vecs[g][e]

            _, srt = plsc.sort_key_val(scores, gix_vm[...], descending=True)
            top8_vm[...] = srt
            idx8 = top8_vm.at[pl.ds(0, TOPK)]
            ck = pltpu.async_copy(k_hbm.at[idx8], kbuf, semK)
            cv = pltpu.async_copy(v_hbm.at[idx8], vbuf, semV)
            ck.wait()

            q_vecs = [qblock[ii, pl.ds(g * LANES, LANES)]
                      for g in range(D // LANES)]
            ws = []
            for h in range(H):
                s8 = jnp.zeros((LANES,))
                for c in range(DH):
                    cc = h * DH + c
                    kcol = plsc.load_gather(
                        kbuf, [lanemod8, jnp.full((LANES,), cc, jnp.int32)])
                    s8 = s8 + kcol * q_vecs[cc // LANES][cc % LANES]
                s8 = s8 * inv_sqrt_dh
                mx = _bmax(jnp.where(lane8, s8, -BIG))
                e = jnp.where(lane8, jnp.exp(s8 - mx), 0.0)
                ws.append(e / _bsum(e))
            cv.wait()

            for h in range(H):
                for blk in range(DH // LANES):
                    off = h * DH + blk * LANES
                    acc = jnp.zeros((LANES,))
                    for kk in range(TOPK):
                        acc = acc + ws[h][kk] * vbuf[kk, pl.ds(off, LANES)]
                    outbuf[pl.ds(off, LANES)] = acc
            pltpu.sync_copy(outbuf, out_hbm.at[r])

        def pair_body(p, _carry):
            r0 = base + 2 * p
            pltpu.async_copy(d2_hbm.at[r0 + 1], d2rowB, semB)
            pltpu.make_async_copy(d2_hbm.at[r0], d2rowA, semA).wait()
            process(r0, 2 * p, d2rowA)
            r2 = jnp.minimum(r0 + 2, base + rows_per_w - 1)
            pltpu.async_copy(d2_hbm.at[r2], d2rowA, semA)
            pltpu.make_async_copy(d2_hbm.at[r0 + 1], d2rowB, semB).wait()
            process(r0 + 1, 2 * p + 1, d2rowB)
            return _carry

        lax.fori_loop(0, rows_per_w // 2, pair_body, 0)
        pltpu.make_async_copy(d2_hbm.at[base], d2rowA, semA).wait()

    return run(d2, aproj, nproj, qarr, karr, varr, coords, w1p, w2)


@jax.jit
def kernel(voxel_tokens, voxel_coords, mask, es_w1, es_b1, es_w2, es_b2,
           wq, bq, wk, bk, wv, bv, wo, bo):
    B, V, _ = voxel_tokens.shape
    R = B * V
    coords_t = jnp.swapaxes(voxel_coords, 1, 2)
    d2 = _d2_call(coords_t).reshape(R, V)

    x2 = voxel_tokens.reshape(R, D)
    wcat = jnp.concatenate(
        [es_w1[:D], es_w1[D:2 * D], wq, wk, wv], axis=1)
    bcat = jnp.concatenate(
        [es_b1, jnp.zeros((64,), jnp.float32), bq, bk, bv]).reshape(1, 512)
    aproj, nproj, qarr, karr, varr = _proj_call(x2, wcat, bcat)

    def _rnd_bf16(x):
        u = lax.bitcast_convert_type(x, jnp.int32)
        r = (u + 0x7FFF + ((u >> 16) & 1)) & jnp.int32(-65536)
        return lax.bitcast_convert_type(r, jnp.float32)
    w1p = _rnd_bf16(es_w1[2 * D:])
    w2 = _rnd_bf16(es_w2[:, 0])
    attn = _sc_sparse(d2, aproj, nproj, qarr, karr, varr,
                      voxel_coords, w1p, w2)

    maskf = mask.reshape(R, 1).astype(jnp.float32)
    out = _outp_call(attn, wo, bo.reshape(1, D), maskf)
    return out.reshape(B, V, D)


# speedup vs baseline: 1.5115x; 1.0584x over previous
import functools

import jax
import jax.numpy as jnp
from jax import lax
from jax.experimental import pallas as pl
from jax.experimental.pallas import tpu as pltpu
from jax.experimental.pallas import tpu_sc as plsc

D = 128
H = 4
DH = 32
KNN = 16
TOPK = 8
NC, NS, LANES = 2, 16, 16
NW = NC * NS
BIG = 3.0e38


def _shuf(x, s):
    perm = jnp.arange(LANES, dtype=jnp.int32) ^ s
    return x.at[perm].get(mode="promise_in_bounds")


def _bmin(x):
    for s in (8, 4, 2, 1):
        x = jnp.minimum(x, _shuf(x, s))
    return x


def _bmax(x):
    for s in (8, 4, 2, 1):
        x = jnp.maximum(x, _shuf(x, s))
    return x


def _bsum(x):
    for s in (8, 4, 2, 1):
        x = x + _shuf(x, s)
    return x


def _bf16r(x):
    u = plsc.bitcast(x, jnp.int32)
    r = (u + 0x7FFF + ((u >> 16) & 1)) & jnp.int32(-65536)
    return plsc.bitcast(r, jnp.float32)


def _d2_body(cfull_ref, crow_ref, out_ref):
    c = cfull_ref[0]
    cr = crow_ref[0]
    sq = jnp.sum(c * c, axis=0, keepdims=True)
    sqr = jnp.sum(cr * cr, axis=0, keepdims=True)
    dot = lax.dot_general(cr.astype(jnp.bfloat16), c.astype(jnp.bfloat16),
                          (((0,), (0,)), ((), ())),
                          preferred_element_type=jnp.float32)
    d2 = sqr.reshape(-1, 1) + sq - 2.0 * dot
    out_ref[0] = jnp.sqrt(jnp.maximum(d2, 1e-12))


def _d2_call(coords_t, BR=256):
    B, _, V = coords_t.shape
    return pl.pallas_call(
        _d2_body,
        grid=(B, V // BR),
        in_specs=[
            pl.BlockSpec((1, 3, V), lambda b, t: (b, 0, 0)),
            pl.BlockSpec((1, 3, BR), lambda b, t: (b, 0, t)),
        ],
        out_specs=pl.BlockSpec((1, BR, V), lambda b, t: (b, t, 0)),
        out_shape=jax.ShapeDtypeStruct((B, V, V), jnp.float32),
    )(coords_t, coords_t)


def _proj_body(x_ref, w_ref, b_ref, a_ref, n_ref, q_ref, k_ref, v_ref):
    y = lax.dot_general(x_ref[...].astype(jnp.bfloat16),
                        w_ref[...].astype(jnp.bfloat16),
                        (((1,), (0,)), ((), ())),
                        preferred_element_type=jnp.float32)
    y = y + b_ref[...]
    a_ref[...] = y[:, 0:64]
    n_ref[...] = y[:, 64:128]
    q_ref[...] = y[:, 128:256]
    k_ref[...] = y[:, 256:384]
    v_ref[...] = y[:, 384:512]


def _proj_call(x2, wcat, bcat, BR=512):
    R = x2.shape[0]
    outs = [
        jax.ShapeDtypeStruct((R, 64), jnp.float32),
        jax.ShapeDtypeStruct((R, 64), jnp.float32),
        jax.ShapeDtypeStruct((R, D), jnp.float32),
        jax.ShapeDtypeStruct((R, D), jnp.float32),
        jax.ShapeDtypeStruct((R, D), jnp.float32),
    ]
    return pl.pallas_call(
        _proj_body,
        grid=(R // BR,),
        in_specs=[
            pl.BlockSpec((BR, D), lambda i: (i, 0)),
            pl.BlockSpec((D, 512), lambda i: (0, 0)),
            pl.BlockSpec((1, 512), lambda i: (0, 0)),
        ],
        out_specs=[
            pl.BlockSpec((BR, 64), lambda i: (i, 0)),
            pl.BlockSpec((BR, 64), lambda i: (i, 0)),
            pl.BlockSpec((BR, D), lambda i: (i, 0)),
            pl.BlockSpec((BR, D), lambda i: (i, 0)),
            pl.BlockSpec((BR, D), lambda i: (i, 0)),
        ],
        out_shape=outs,
    )(x2, wcat, bcat)


def _outp_body(x_ref, w_ref, b_ref, m_ref, o_ref):
    y = lax.dot_general(x_ref[...], w_ref[...], (((1,), (0,)), ((), ())))
    o_ref[...] = (y + b_ref[...]) * m_ref[...]


def _outp_call(attn, wo, bo2, maskf, BR=512):
    R = attn.shape[0]
    return pl.pallas_call(
        _outp_body,
        grid=(R // BR,),
        in_specs=[
            pl.BlockSpec((BR, D), lambda i: (i, 0)),
            pl.BlockSpec((D, D), lambda i: (0, 0)),
            pl.BlockSpec((1, D), lambda i: (0, 0)),
            pl.BlockSpec((BR, 1), lambda i: (i, 0)),
        ],
        out_specs=pl.BlockSpec((BR, D), lambda i: (i, 0)),
        out_shape=jax.ShapeDtypeStruct((R, D), jnp.float32),
    )(attn, wo, bo2, maskf)


def _sc_sparse(d2, aproj, nproj, qarr, karr, varr, coords, w1p, w2):
    R, V = d2.shape
    B = coords.shape[0]
    rows_per_w = R // NW
    rows_per_b = R // B
    nblk = V // LANES
    mesh = plsc.VectorSubcoreMesh(
        core_axis_name="c", subcore_axis_name="s",
        num_cores=NC, num_subcores=NS)

    @functools.partial(
        pl.kernel,
        out_type=jax.ShapeDtypeStruct((R, D), jnp.float32),
        mesh=mesh,
        compiler_params=pltpu.CompilerParams(
            needs_layout_passes=False, use_tc_tiling_on_sc=False),
        scratch_types=[
            pltpu.VMEM((V,), jnp.float32),
            pltpu.VMEM((V,), jnp.float32),
            pltpu.VMEM((V, 3), jnp.float32),
            pltpu.VMEM((R // NW, 64), jnp.float32),
            pltpu.VMEM((R // NW, D), jnp.float32),
            pltpu.VMEM((KNN, 64), jnp.float32),
            pltpu.VMEM((TOPK, D), jnp.float32),
            pltpu.VMEM((TOPK, D), jnp.float32),
            pltpu.VMEM((32,), jnp.int32),
            pltpu.VMEM((KNN,), jnp.int32),
            pltpu.VMEM((KNN,), jnp.int32),
            pltpu.VMEM((D,), jnp.float32),
            pltpu.VMEM((3, 64), jnp.float32),
            pltpu.VMEM((64,), jnp.float32),
            pltpu.SemaphoreType.DMA,
            pltpu.SemaphoreType.DMA,
            pltpu.SemaphoreType.DMA,
            pltpu.SemaphoreType.DMA,
            pltpu.SemaphoreType.DMA,
        ],
    )
    def run(d2_hbm, a_hbm, n_hbm, q_hbm, k_hbm, v_hbm, c_hbm, w1p_hbm,
            w2_hbm, out_hbm, d2rowA, d2rowB, coords_vm, ablock, qblock,
            nbuf, kbuf, vbuf, knnbuf, gix_vm, top8_vm, outbuf, w1p_vm,
            w2_vm, semA, semB, semN, semK, semV):
        wid = lax.axis_index("s") * NC + lax.axis_index("c")
        base = wid * rows_per_w
        b = base // rows_per_b
        lane = lax.iota(jnp.int32, LANES)
        lane0 = lane == 0
        lane8 = lane < TOPK
        lanemod8 = lane & 7
        inv_sqrt_dh = jnp.float32(1.0 / (DH ** 0.5))

        pltpu.async_copy(d2_hbm.at[base], d2rowA, semA)
        pltpu.sync_copy(c_hbm.at[b], coords_vm)
        pltpu.sync_copy(w1p_hbm, w1p_vm)
        pltpu.sync_copy(w2_hbm, w2_vm)
        pltpu.sync_copy(a_hbm.at[pl.ds(base, rows_per_w)], ablock)
        pltpu.sync_copy(q_hbm.at[pl.ds(base, rows_per_w)], qblock)

        w2_vecs = [w2_vm[pl.ds(g * LANES, LANES)] for g in range(4)]
        wpx = [w1p_vm[0, pl.ds(g * LANES, LANES)] for g in range(4)]
        wpy = [w1p_vm[1, pl.ds(g * LANES, LANES)] for g in range(4)]
        wpz = [w1p_vm[2, pl.ds(g * LANES, LANES)] for g in range(4)]

        def process(r, ii, d2row):
            i = r - b * rows_per_b

            def fold(v, carry):
                col, colv = carry
                blk = d2row[pl.ds(pl.multiple_of(v * LANES, LANES), LANES)]
                lt = blk < col
                return jnp.where(lt, blk, col), jnp.where(lt, v, colv)
            col, colv = lax.fori_loop(
                0, nblk, fold,
                (jnp.full((LANES,), BIG), jnp.zeros((LANES,), jnp.int32)),
                unroll=8)

            def rnd(rd, carry):
                col, colv = carry
                m = _bmin(col)
                jcand = jnp.where(col == m, colv * LANES + lane,
                                  jnp.int32(1 << 30))
                j = _bmin(jcand)
                slot = jnp.where(rd == 0, 16, rd - 1)
                plsc.store_scatter(knnbuf, [jnp.broadcast_to(slot, (LANES,))],
                                   j, mask=lane0)
                plsc.store_scatter(d2row, [j], jnp.full((LANES,), BIG),
                                   mask=lane0)
                lstar = j & (LANES - 1)
                base_jj = lane * LANES + lstar
                gs = [plsc.load_gather(d2row, [t * 256 + base_jj])
                      for t in range(nblk // LANES)]
                cmv = gs[0]
                for t in range(1, nblk // LANES):
                    cmv = jnp.minimum(cmv, gs[t])
                colmin = _bmin(cmv)
                bestv = jnp.full((LANES,), 4096, jnp.int32)
                for t in range(nblk // LANES):
                    first = plsc.all_reduce_ffs(gs[t] == colmin)
                    bestv = jnp.where((bestv >= 4096) & (first < LANES),
                                      t * LANES + first, bestv)
                sel = lane == lstar
                return (jnp.where(sel, colmin, col),
                        jnp.where(sel, bestv, colv))
            col, colv = lax.fori_loop(0, KNN, rnd, (col, colv))
            m = _bmin(col)
            jcand = jnp.where(col == m, colv * LANES + lane, jnp.int32(1 << 30))
            j = _bmin(jcand)
            plsc.store_scatter(knnbuf, [jnp.full((LANES,), 15, jnp.int32)],
                               j, mask=lane0)

            knn = knnbuf[pl.ds(0, KNN)]
            gix_vm[...] = knn + b * rows_per_b

            cn = pltpu.async_copy(n_hbm.at[gix_vm], nbuf, semN)

            ivec = jnp.broadcast_to(i, (LANES,))
            nx = plsc.load_gather(coords_vm, [knn, jnp.zeros((LANES,), jnp.int32)])
            ny = plsc.load_gather(coords_vm, [knn, jnp.full((LANES,), 1, jnp.int32)])
            nz = plsc.load_gather(coords_vm, [knn, jnp.full((LANES,), 2, jnp.int32)])
            dx = _bf16r(nx - plsc.load_gather(coords_vm, [ivec, jnp.zeros((LANES,), jnp.int32)]))
            dy = _bf16r(ny - plsc.load_gather(coords_vm, [ivec, jnp.full((LANES,), 1, jnp.int32)]))
            dz = _bf16r(nz - plsc.load_gather(coords_vm, [ivec, jnp.full((LANES,), 2, jnp.int32)]))
            a_vecs = [ablock[ii, pl.ds(g * LANES, LANES)] for g in range(4)]
            rps = []
            for c in range(64):
                g, e = c // LANES, c % LANES
                rps.append(dx * wpx[g][e] + dy * wpy[g][e] + dz * wpz[g][e])
            cn.wait()

            scores = jnp.zeros((LANES,))
            for c in range(64):
                g, e = c // LANES, c % LANES
                ncol = plsc.load_gather(nbuf, [lane, jnp.full((LANES,), c, jnp.int32)])
                hh = jnp.maximum(ncol + rps[c] + a_vecs[g][e], 0.0)
                scores = scores + _bf16r(hh) * w2_vecs[g][e]

            _, srt = plsc.sort_key_val(scores, gix_vm[...], descending=True)
            top8_vm[...] = srt
            idx8 = top8_vm.at[pl.ds(0, TOPK)]
            ck = pltpu.async_copy(k_hbm.at[idx8], kbuf, semK)
            cv = pltpu.async_copy(v_hbm.at[idx8], vbuf, semV)
            ck.wait()

            q_vecs = [qblock[ii, pl.ds(g * LANES, LANES)]
                      for g in range(D // LANES)]
            ws = []
            for h in range(H):
                s8 = jnp.zeros((LANES,))
                for c in range(DH):
                    cc = h * DH + c
                    kcol = plsc.load_gather(
                        kbuf, [lanemod8, jnp.full((LANES,), cc, jnp.int32)])
                    s8 = s8 + kcol * q_vecs[cc // LANES][cc % LANES]
                s8 = s8 * inv_sqrt_dh
                mx = _bmax(jnp.where(lane8, s8, -BIG))
                e = jnp.where(lane8, jnp.exp(s8 - mx), 0.0)
                ws.append(e / _bsum(e))
            cv.wait()

            for h in range(H):
                for blk in range(DH // LANES):
                    off = h * DH + blk * LANES
                    acc = jnp.zeros((LANES,))
                    for kk in range(TOPK):
                        acc = acc + ws[h][kk] * vbuf[kk, pl.ds(off, LANES)]
                    outbuf[pl.ds(off, LANES)] = acc
            pltpu.sync_copy(outbuf, out_hbm.at[r])

        def pair_body(p, _carry):
            r0 = base + 2 * p
            pltpu.async_copy(d2_hbm.at[r0 + 1], d2rowB, semB)
            pltpu.make_async_copy(d2_hbm.at[r0], d2rowA, semA).wait()
            process(r0, 2 * p, d2rowA)
            r2 = jnp.minimum(r0 + 2, base + rows_per_w - 1)
            pltpu.async_copy(d2_hbm.at[r2], d2rowA, semA)
            pltpu.make_async_copy(d2_hbm.at[r0 + 1], d2rowB, semB).wait()
            process(r0 + 1, 2 * p + 1, d2rowB)
            return _carry

        lax.fori_loop(0, rows_per_w // 2, pair_body, 0)
        pltpu.make_async_copy(d2_hbm.at[base], d2rowA, semA).wait()

    return run(d2, aproj, nproj, qarr, karr, varr, coords, w1p, w2)


@jax.jit
def kernel(voxel_tokens, voxel_coords, mask, es_w1, es_b1, es_w2, es_b2,
           wq, bq, wk, bk, wv, bv, wo, bo):
    B, V, _ = voxel_tokens.shape
    R = B * V
    coords_t = jnp.swapaxes(voxel_coords, 1, 2)
    d2 = _d2_call(coords_t).reshape(R, V)

    x2 = voxel_tokens.reshape(R, D)
    wcat = jnp.concatenate(
        [es_w1[:D], es_w1[D:2 * D], wq, wk, wv], axis=1)
    bcat = jnp.concatenate(
        [es_b1, jnp.zeros((64,), jnp.float32), bq, bk, bv]).reshape(1, 512)
    aproj, nproj, qarr, karr, varr = _proj_call(x2, wcat, bcat)

    def _rnd_bf16(x):
        u = lax.bitcast_convert_type(x, jnp.int32)
        r = (u + 0x7FFF + ((u >> 16) & 1)) & jnp.int32(-65536)
        return lax.bitcast_convert_type(r, jnp.float32)
    w1p = _rnd_bf16(es_w1[2 * D:])
    w2 = _rnd_bf16(es_w2[:, 0])
    attn = _sc_sparse(d2, aproj, nproj, qarr, karr, varr,
                      voxel_coords, w1p, w2)

    maskf = mask.reshape(R, 1).astype(jnp.float32)
    out = _outp_call(attn, wo, bo.reshape(1, D), maskf)
    return out.reshape(B, V, D)
